# baseline (speedup 1.0000x reference)
import functools

import jax
import jax.numpy as jnp
from jax import lax
from jax.experimental import pallas as pl
from jax.experimental.pallas import tpu as pltpu
from jax.experimental.pallas import tpu_sc as plsc

N = 10000
K = 32
LATENT = 512
SCALE_GRAD = 0.4 / N
SCALE_E = 0.1 / N

NW = 32
VPW = 320
NPAD = NW * VPW
M = 3 * N
MPAD = 3 * NPAD
SLAB = K * VPW
TILE = 2048
GRID = MPAD // TILE

_mesh = plsc.VectorSubcoreMesh(core_axis_name="c", subcore_axis_name="s")
_sc_params = pltpu.CompilerParams(needs_layout_passes=False)


def _fwd_body(code_ref, w_ref, b_ref, out_ref):
    t = pl.program_id(0)
    r = jnp.dot(code_ref[...], w_ref[...], preferred_element_type=jnp.float32)
    r = r + b_ref[...]
    col = t * TILE + lax.broadcasted_iota(jnp.int32, (1, TILE), 1)
    out_ref[...] = jnp.where(col < M, r, 0.0)


_fwd_call = pl.pallas_call(
    _fwd_body,
    grid=(GRID,),
    in_specs=[
        pl.BlockSpec((1, LATENT), lambda t: (0, 0)),
        pl.BlockSpec((LATENT, TILE), lambda t: (0, t)),
        pl.BlockSpec((1, TILE), lambda t: (0, t)),
    ],
    out_specs=pl.BlockSpec((1, TILE), lambda t: (0, t)),
    out_shape=jax.ShapeDtypeStruct((1, MPAD), jnp.float32),
)


@functools.partial(
    pl.kernel,
    out_type=[
        jax.ShapeDtypeStruct((NW, SLAB), jnp.float32),
        jax.ShapeDtypeStruct((NW, SLAB), jnp.int32),
        jax.ShapeDtypeStruct((NW, SLAB), jnp.float32),
    ],
    mesh=_mesh,
    scratch_types=[
        pltpu.VMEM((MPAD,), jnp.float32),
        pltpu.VMEM((SLAB,), jnp.int32),
        pltpu.VMEM((SLAB,), jnp.float32),
        pltpu.VMEM((VPW,), jnp.int32),
        pltpu.VMEM((VPW,), jnp.float32),
        pltpu.VMEM((SLAB,), jnp.float32),
        pltpu.VMEM((SLAB,), jnp.int32),
        pltpu.VMEM((SLAB,), jnp.float32),
    ],
    compiler_params=_sc_params,
)
def _pre_call(xyz_hbm, nbr_hbm, w_hbm, nn_hbm, area_hbm,
              s0_hbm, nbrt_hbm, wt_hbm,
              xyz_v, nbr_v, w_v, nn_v, area_v, s0_v, nbrt_v, wt_v):
    wid = lax.axis_index("s") * 2 + lax.axis_index("c")

    pltpu.sync_copy(xyz_hbm, xyz_v)
    pltpu.sync_copy(nbr_hbm.at[wid], nbr_v)
    pltpu.sync_copy(w_hbm.at[wid], w_v)
    pltpu.sync_copy(nn_hbm.at[wid], nn_v)
    pltpu.sync_copy(area_hbm.at[wid], area_v)

    iota16 = lax.iota(jnp.int32, 16)

    def _block(b, carry):
        v0 = b * 16
        g0 = wid * VPW + v0
        sidx = 3 * g0 + 3 * iota16
        px = plsc.load_gather(xyz_v, [sidx])
        py = plsc.load_gather(xyz_v, [sidx + 1])
        pz = plsc.load_gather(xyz_v, [sidx + 2])
        nnv = nn_v[pl.ds(v0, 16)]
        areav = area_v[pl.ds(v0, 16)]
        rowbase = (v0 + iota16) * K
        for k in range(K):
            nbr = plsc.load_gather(nbr_v, [rowbase + k])
            w = plsc.load_gather(w_v, [rowbase + k])
            wf = jnp.where(nnv > k, w * areav, 0.0)
            jb = nbr * 3
            qx = plsc.load_gather(xyz_v, [jb])
            qy = plsc.load_gather(xyz_v, [jb + 1])
            qz = plsc.load_gather(xyz_v, [jb + 2])
            dx = px - qx
            dy = py - qy
            dz = pz - qz
            s0 = dx * dx + dy * dy + dz * dz
            s0_v[pl.ds(k * VPW + v0, 16)] = s0
            nbrt_v[pl.ds(k * VPW + v0, 16)] = nbr
            wt_v[pl.ds(k * VPW + v0, 16)] = wf
        return carry

    lax.fori_loop(0, VPW // 16, _block, 0)
    pltpu.sync_copy(s0_v, s0_hbm.at[wid])
    pltpu.sync_copy(nbrt_v, nbrt_hbm.at[wid])
    pltpu.sync_copy(wt_v, wt_hbm.at[wid])


@functools.partial(
    pl.kernel,
    out_type=[
        jax.ShapeDtypeStruct((NW, MPAD), jnp.float32),
        jax.ShapeDtypeStruct((NW, 16), jnp.float32),
    ],
    mesh=_mesh,
    scratch_types=[
        pltpu.VMEM((MPAD,), jnp.float32),
        pltpu.VMEM((MPAD,), jnp.float32),
        pltpu.VMEM((SLAB,), jnp.float32),
        pltpu.VMEM((SLAB,), jnp.int32),
        pltpu.VMEM((SLAB,), jnp.float32),
        pltpu.VMEM((16,), jnp.float32),
    ],
    compiler_params=_sc_params,
)
def _edge_call(recon_hbm, s0_hbm, nbr_hbm, w_hbm,
               gpart_hbm, epart_hbm,
               recon_v, grad_v, s0_v, nbr_v, w_v, e_v):
    wid = lax.axis_index("s") * 2 + lax.axis_index("c")

    pltpu.sync_copy(recon_hbm, recon_v)
    pltpu.sync_copy(s0_hbm.at[wid], s0_v)
    pltpu.sync_copy(nbr_hbm.at[wid], nbr_v)
    pltpu.sync_copy(w_hbm.at[wid], w_v)

    zeros16 = jnp.zeros((16,), jnp.float32)

    def _zero(z, _):
        base = z * 256
        for u in range(16):
            grad_v[pl.ds(base + u * 16, 16)] = zeros16
        return 0

    lax.fori_loop(0, MPAD // 256, _zero, 0)

    iota16 = lax.iota(jnp.int32, 16)

    def _block(b, eacc):
        v0 = b * 16
        g0 = wid * VPW + v0
        sidx = 3 * g0 + 3 * iota16
        sx = plsc.load_gather(recon_v, [sidx])
        sy = plsc.load_gather(recon_v, [sidx + 1])
        sz = plsc.load_gather(recon_v, [sidx + 2])

        gx = zeros16
        gy = zeros16
        gz = zeros16
        ek = zeros16
        for k in range(K):
            o = k * VPW + v0
            nbr = nbr_v[pl.ds(o, 16)]
            wf = w_v[pl.ds(o, 16)]
            s0 = s0_v[pl.ds(o, 16)]
            jb = nbr * 3
            rx = plsc.load_gather(recon_v, [jb])
            ry = plsc.load_gather(recon_v, [jb + 1])
            rz = plsc.load_gather(recon_v, [jb + 2])
            e1x = sx - rx
            e1y = sy - ry
            e1z = sz - rz
            d = (e1x * e1x + e1y * e1y + e1z * e1z) - s0
            wmd = wf * d
            ek = ek + wmd * d
            q = wmd * SCALE_GRAD
            cx = q * e1x
            cy = q * e1y
            cz = q * e1z
            gx = gx + cx
            gy = gy + cy
            gz = gz + cz
            plsc.addupdate_scatter(grad_v, [jb], -cx)
            plsc.addupdate_scatter(grad_v, [jb + 1], -cy)
            plsc.addupdate_scatter(grad_v, [jb + 2], -cz)

        plsc.addupdate_scatter(grad_v, [sidx], gx)
        plsc.addupdate_scatter(grad_v, [sidx + 1], gy)
        plsc.addupdate_scatter(grad_v, [sidx + 2], gz)
        return eacc + ek

    eacc = lax.fori_loop(0, VPW // 16, _block, zeros16)
    e_v[...] = eacc
    pltpu.sync_copy(grad_v, gpart_hbm.at[wid])
    pltpu.sync_copy(e_v, epart_hbm.at[wid])


def _bwd_body(w_ref, gp_ref, ep_ref, gc_ref, e_ref):
    t = pl.program_id(0)

    @pl.when(t == 0)
    def _():
        gc_ref[...] = jnp.zeros_like(gc_ref)
        e_ref[...] = (jnp.sum(ep_ref[...]) * SCALE_E).reshape(1, 1)

    col = t * TILE + lax.broadcasted_iota(jnp.int32, (1, TILE), 1)
    wm = jnp.where(col < M, w_ref[...], 0.0)
    g = jnp.sum(gp_ref[...], axis=0, keepdims=True)
    contrib = lax.dot_general(g, wm, (((1,), (1,)), ((), ())),
                              preferred_element_type=jnp.float32)
    gc_ref[...] += contrib


_bwd_call = pl.pallas_call(
    _bwd_body,
    grid=(GRID,),
    in_specs=[
        pl.BlockSpec((LATENT, TILE), lambda t: (0, t)),
        pl.BlockSpec((NW, TILE), lambda t: (0, t)),
        pl.BlockSpec((NW, 16), lambda t: (0, 0)),
    ],
    out_specs=[
        pl.BlockSpec((1, LATENT), lambda t: (0, 0)),
        pl.BlockSpec((1, 1), lambda t: (0, 0)),
    ],
    out_shape=[
        jax.ShapeDtypeStruct((1, LATENT), jnp.float32),
        jax.ShapeDtypeStruct((1, 1), jnp.float32),
    ],
)


def kernel(code, W_dec, b_dec, xyz1, neighbors, num_neighbors, weights, area):
    xyzf = jnp.pad(xyz1.reshape(M), (0, MPAD - M))
    nbrP = jnp.pad(neighbors.astype(jnp.int32), ((0, NPAD - N), (0, 0))) \
        .reshape(NW, SLAB)
    wP = jnp.pad(weights, ((0, NPAD - N), (0, 0))).reshape(NW, SLAB)
    nnP = jnp.pad(num_neighbors.astype(jnp.int32), (0, NPAD - N)) \
        .reshape(NW, VPW)
    areaP = jnp.pad(area, (0, NPAD - N)).reshape(NW, VPW)

    s0, nbrT, wT = _pre_call(xyzf, nbrP, wP, nnP, areaP)

    b_pad = jnp.pad(b_dec, (0, MPAD - M)).reshape(1, MPAD)
    recon = _fwd_call(code.reshape(1, LATENT), W_dec, b_pad).reshape(MPAD)

    gpart, epart = _edge_call(recon, s0, nbrT, wT)

    gc, e = _bwd_call(W_dec, gpart, epart)
    return e[0, 0], gc[0]


# speedup vs baseline: 1.0699x; 1.0699x over previous
import functools

import jax
import jax.numpy as jnp
from jax import lax
from jax.experimental import pallas as pl
from jax.experimental.pallas import tpu as pltpu
from jax.experimental.pallas import tpu_sc as plsc

N = 10000
K = 32
LATENT = 512
SCALE_GRAD = 0.4 / N
SCALE_E = 0.1 / N

NW = 32
VPW = 320
NPAD = NW * VPW
M = 3 * N
MPAD = 3 * NPAD
SLAB = K * VPW
NK = N * K
TILE = 2048
GRID = MPAD // TILE

LASTW = NW - 1
TAILV = N - LASTW * VPW
TAILE = TAILV * K

_mesh = plsc.VectorSubcoreMesh(core_axis_name="c", subcore_axis_name="s")
_sc_params = pltpu.CompilerParams(needs_layout_passes=False)


def _fwd_body(code_ref, w_ref, b_ref, out_ref):
    t = pl.program_id(0)
    r = jnp.dot(code_ref[...], w_ref[...], preferred_element_type=jnp.float32)
    r = r + b_ref[...]
    col = t * TILE + lax.broadcasted_iota(jnp.int32, (1, TILE), 1)
    out_ref[...] = jnp.where(col < M, r, 0.0)


_fwd_call = pl.pallas_call(
    _fwd_body,
    grid=(GRID,),
    in_specs=[
        pl.BlockSpec((1, LATENT), lambda t: (0, 0)),
        pl.BlockSpec((LATENT, TILE), lambda t: (0, t)),
        pl.BlockSpec((1, TILE), lambda t: (0, t)),
    ],
    out_specs=pl.BlockSpec((1, TILE), lambda t: (0, t)),
    out_shape=jax.ShapeDtypeStruct((1, MPAD), jnp.float32),
)


@functools.partial(
    pl.kernel,
    out_type=[
        jax.ShapeDtypeStruct((NW, SLAB), jnp.float32),
        jax.ShapeDtypeStruct((NW, SLAB), jnp.int32),
        jax.ShapeDtypeStruct((NW, SLAB), jnp.float32),
    ],
    mesh=_mesh,
    scratch_types=[
        pltpu.VMEM((MPAD,), jnp.float32),
        pltpu.VMEM((SLAB,), jnp.int32),
        pltpu.VMEM((SLAB,), jnp.float32),
        pltpu.VMEM((VPW,), jnp.int32),
        pltpu.VMEM((VPW,), jnp.float32),
        pltpu.VMEM((SLAB,), jnp.float32),
        pltpu.VMEM((SLAB,), jnp.int32),
        pltpu.VMEM((SLAB,), jnp.float32),
    ],
    compiler_params=_sc_params,
)
def _pre_call(xyz_hbm, nbr_hbm, w_hbm, nn_hbm, area_hbm,
              s0_hbm, nbrt_hbm, wt_hbm,
              xyz_v, nbr_v, w_v, nn_v, area_v, s0_v, nbrt_v, wt_v):
    wid = lax.axis_index("s") * 2 + lax.axis_index("c")

    zeros16 = jnp.zeros((16,), jnp.float32)
    izeros16 = jnp.zeros((16,), jnp.int32)

    pltpu.sync_copy(xyz_hbm, xyz_v.at[pl.ds(0, M)])
    for u in range((MPAD - M) // 16):
        xyz_v[pl.ds(M + u * 16, 16)] = zeros16

    @pl.when(wid != LASTW)
    def _():
        pltpu.sync_copy(nbr_hbm.at[pl.ds(wid * SLAB, SLAB)], nbr_v)
        pltpu.sync_copy(w_hbm.at[pl.ds(wid * SLAB, SLAB)], w_v)
        pltpu.sync_copy(nn_hbm.at[pl.ds(wid * VPW, VPW)], nn_v)
        pltpu.sync_copy(area_hbm.at[pl.ds(wid * VPW, VPW)], area_v)

    @pl.when(wid == LASTW)
    def _():
        pltpu.sync_copy(nbr_hbm.at[pl.ds(LASTW * SLAB, TAILE)],
                        nbr_v.at[pl.ds(0, TAILE)])
        pltpu.sync_copy(w_hbm.at[pl.ds(LASTW * SLAB, TAILE)],
                        w_v.at[pl.ds(0, TAILE)])
        pltpu.sync_copy(nn_hbm.at[pl.ds(LASTW * VPW, TAILV)],
                        nn_v.at[pl.ds(0, TAILV)])
        pltpu.sync_copy(area_hbm.at[pl.ds(LASTW * VPW, TAILV)],
                        area_v.at[pl.ds(0, TAILV)])

        def _ztail(z, _):
            nbr_v[pl.ds(TAILE + z * 16, 16)] = izeros16
            return 0

        lax.fori_loop(0, (SLAB - TAILE) // 16, _ztail, 0)
        for u in range((VPW - TAILV) // 16):
            nn_v[pl.ds(TAILV + u * 16, 16)] = izeros16

    iota16 = lax.iota(jnp.int32, 16)

    def _block(b, carry):
        v0 = b * 16
        g0 = wid * VPW + v0
        sidx = 3 * g0 + 3 * iota16
        px = plsc.load_gather(xyz_v, [sidx])
        py = plsc.load_gather(xyz_v, [sidx + 1])
        pz = plsc.load_gather(xyz_v, [sidx + 2])
        nnv = nn_v[pl.ds(v0, 16)]
        areav = area_v[pl.ds(v0, 16)]
        rowbase = (v0 + iota16) * K
        for k in range(K):
            nbr = plsc.load_gather(nbr_v, [rowbase + k])
            w = plsc.load_gather(w_v, [rowbase + k])
            wf = jnp.where(nnv > k, w * areav, 0.0)
            jb = nbr * 3
            qx = plsc.load_gather(xyz_v, [jb])
            qy = plsc.load_gather(xyz_v, [jb + 1])
            qz = plsc.load_gather(xyz_v, [jb + 2])
            dx = px - qx
            dy = py - qy
            dz = pz - qz
            s0 = dx * dx + dy * dy + dz * dz
            s0_v[pl.ds(k * VPW + v0, 16)] = s0
            nbrt_v[pl.ds(k * VPW + v0, 16)] = nbr
            wt_v[pl.ds(k * VPW + v0, 16)] = wf
        return carry

    lax.fori_loop(0, VPW // 16, _block, 0)
    pltpu.sync_copy(s0_v, s0_hbm.at[wid])
    pltpu.sync_copy(nbrt_v, nbrt_hbm.at[wid])
    pltpu.sync_copy(wt_v, wt_hbm.at[wid])


@functools.partial(
    pl.kernel,
    out_type=[
        jax.ShapeDtypeStruct((NW, MPAD), jnp.float32),
        jax.ShapeDtypeStruct((NW, 16), jnp.float32),
    ],
    mesh=_mesh,
    scratch_types=[
        pltpu.VMEM((MPAD,), jnp.float32),
        pltpu.VMEM((MPAD,), jnp.float32),
        pltpu.VMEM((SLAB,), jnp.float32),
        pltpu.VMEM((SLAB,), jnp.int32),
        pltpu.VMEM((SLAB,), jnp.float32),
        pltpu.VMEM((16,), jnp.float32),
    ],
    compiler_params=_sc_params,
)
def _edge_call(recon_hbm, s0_hbm, nbr_hbm, w_hbm,
               gpart_hbm, epart_hbm,
               recon_v, grad_v, s0_v, nbr_v, w_v, e_v):
    wid = lax.axis_index("s") * 2 + lax.axis_index("c")

    zeros16 = jnp.zeros((16,), jnp.float32)

    pltpu.sync_copy(recon_hbm, recon_v)
    pltpu.sync_copy(s0_hbm.at[wid], s0_v)
    pltpu.sync_copy(nbr_hbm.at[wid], nbr_v)
    pltpu.sync_copy(w_hbm.at[wid], w_v)

    def _zero(z, _):
        base = z * 256
        for u in range(16):
            grad_v[pl.ds(base + u * 16, 16)] = zeros16
        return 0

    lax.fori_loop(0, MPAD // 256, _zero, 0)

    iota16 = lax.iota(jnp.int32, 16)

    def _block(b, eacc):
        v0 = b * 16
        g0 = wid * VPW + v0
        sidx = 3 * g0 + 3 * iota16
        sx = plsc.load_gather(recon_v, [sidx])
        sy = plsc.load_gather(recon_v, [sidx + 1])
        sz = plsc.load_gather(recon_v, [sidx + 2])

        gx = zeros16
        gy = zeros16
        gz = zeros16
        ek = zeros16
        for k in range(K):
            o = k * VPW + v0
            nbr = nbr_v[pl.ds(o, 16)]
            wf = w_v[pl.ds(o, 16)]
            s0 = s0_v[pl.ds(o, 16)]
            jb = nbr * 3
            rx = plsc.load_gather(recon_v, [jb])
            ry = plsc.load_gather(recon_v, [jb + 1])
            rz = plsc.load_gather(recon_v, [jb + 2])
            e1x = sx - rx
            e1y = sy - ry
            e1z = sz - rz
            d = (e1x * e1x + e1y * e1y + e1z * e1z) - s0
            wmd = wf * d
            ek = ek + wmd * d
            q = wmd * SCALE_GRAD
            cx = q * e1x
            cy = q * e1y
            cz = q * e1z
            gx = gx + cx
            gy = gy + cy
            gz = gz + cz
            plsc.addupdate_scatter(grad_v, [jb], -cx)
            plsc.addupdate_scatter(grad_v, [jb + 1], -cy)
            plsc.addupdate_scatter(grad_v, [jb + 2], -cz)

        plsc.addupdate_scatter(grad_v, [sidx], gx)
        plsc.addupdate_scatter(grad_v, [sidx + 1], gy)
        plsc.addupdate_scatter(grad_v, [sidx + 2], gz)
        return eacc + ek

    eacc = lax.fori_loop(0, VPW // 16, _block, zeros16)
    e_v[...] = eacc
    pltpu.sync_copy(grad_v, gpart_hbm.at[wid])
    pltpu.sync_copy(e_v, epart_hbm.at[wid])


def _bwd_body(w_ref, gp_ref, ep_ref, gc_ref, e_ref):
    t = pl.program_id(0)

    @pl.when(t == 0)
    def _():
        gc_ref[...] = jnp.zeros_like(gc_ref)
        e_ref[...] = (jnp.sum(ep_ref[...]) * SCALE_E).reshape(1, 1)

    col = t * TILE + lax.broadcasted_iota(jnp.int32, (1, TILE), 1)
    wm = jnp.where(col < M, w_ref[...], 0.0)
    g = jnp.sum(gp_ref[...], axis=0, keepdims=True)
    contrib = lax.dot_general(g, wm, (((1,), (1,)), ((), ())),
                              preferred_element_type=jnp.float32)
    gc_ref[...] += contrib


_bwd_call = pl.pallas_call(
    _bwd_body,
    grid=(GRID,),
    in_specs=[
        pl.BlockSpec((LATENT, TILE), lambda t: (0, t)),
        pl.BlockSpec((NW, TILE), lambda t: (0, t)),
        pl.BlockSpec((NW, 16), lambda t: (0, 0)),
    ],
    out_specs=[
        pl.BlockSpec((1, LATENT), lambda t: (0, 0)),
        pl.BlockSpec((1, 1), lambda t: (0, 0)),
    ],
    out_shape=[
        jax.ShapeDtypeStruct((1, LATENT), jnp.float32),
        jax.ShapeDtypeStruct((1, 1), jnp.float32),
    ],
)


def kernel(code, W_dec, b_dec, xyz1, neighbors, num_neighbors, weights, area):
    xyzf = xyz1.reshape(M)
    nbrF = neighbors.astype(jnp.int32).reshape(NK)
    wF = weights.reshape(NK)
    nnI = num_neighbors.astype(jnp.int32)

    s0, nbrT, wT = _pre_call(xyzf, nbrF, wF, nnI, area)

    b_pad = jnp.pad(b_dec, (0, MPAD - M)).reshape(1, MPAD)
    recon = _fwd_call(code.reshape(1, LATENT), W_dec, b_pad).reshape(MPAD)

    gpart, epart = _edge_call(recon, s0, nbrT, wT)

    gc, e = _bwd_call(W_dec, gpart, epart)
    return e[0, 0], gc[0]


# speedup vs baseline: 1.3903x; 1.2995x over previous
import functools

import jax
import jax.numpy as jnp
from jax import lax
from jax.experimental import pallas as pl
from jax.experimental.pallas import tpu as pltpu
from jax.experimental.pallas import tpu_sc as plsc

N = 10000
K = 32
LATENT = 512
SCALE_GRAD = 0.4 / N
SCALE_E = 0.1 / N

NW = 32
VPW = 320
NPAD = NW * VPW
M = 3 * N
MPAD = 3 * NPAD
SLAB = K * VPW
NK = N * K
TILE = 2048
GRID = MPAD // TILE

LASTW = NW - 1
TAILV = N - LASTW * VPW
TAILE = TAILV * K

_mesh = plsc.VectorSubcoreMesh(core_axis_name="c", subcore_axis_name="s")
_sc_params = pltpu.CompilerParams(needs_layout_passes=False)


def _fwd_body(code_ref, wt_ref, b_ref, out_ref):
    t = pl.program_id(0)
    r = lax.dot_general(code_ref[...], wt_ref[...], (((1,), (1,)), ((), ())),
                        preferred_element_type=jnp.float32)
    r = r + b_ref[...]
    col = t * TILE + lax.broadcasted_iota(jnp.int32, (1, TILE), 1)
    out_ref[...] = jnp.where(col < M, r, 0.0)


_fwd_call = pl.pallas_call(
    _fwd_body,
    grid=(GRID,),
    in_specs=[
        pl.BlockSpec((1, LATENT), lambda t: (0, 0)),
        pl.BlockSpec((TILE, LATENT), lambda t: (t, 0)),
        pl.BlockSpec((1, TILE), lambda t: (0, t)),
    ],
    out_specs=pl.BlockSpec((1, TILE), lambda t: (0, t)),
    out_shape=jax.ShapeDtypeStruct((1, MPAD), jnp.float32),
)


@functools.partial(
    pl.kernel,
    out_type=[
        jax.ShapeDtypeStruct((NW, SLAB), jnp.float32),
        jax.ShapeDtypeStruct((NW, SLAB), jnp.int32),
        jax.ShapeDtypeStruct((NW, SLAB), jnp.float32),
    ],
    mesh=_mesh,
    scratch_types=[
        pltpu.VMEM((MPAD,), jnp.float32),
        pltpu.VMEM((SLAB,), jnp.int32),
        pltpu.VMEM((SLAB,), jnp.float32),
        pltpu.VMEM((VPW,), jnp.int32),
        pltpu.VMEM((VPW,), jnp.float32),
        pltpu.VMEM((SLAB,), jnp.float32),
        pltpu.VMEM((SLAB,), jnp.int32),
        pltpu.VMEM((SLAB,), jnp.float32),
    ],
    compiler_params=_sc_params,
)
def _pre_call(xyz_hbm, nbr_hbm, w_hbm, nn_hbm, area_hbm,
              s0_hbm, nbro_hbm, wo_hbm,
              xyz_v, nbr_v, w_v, nn_v, area_v, s0_v, nbrt_v, wf_v):
    wid = lax.axis_index("s") * 2 + lax.axis_index("c")

    zeros16 = jnp.zeros((16,), jnp.float32)
    izeros16 = jnp.zeros((16,), jnp.int32)

    pltpu.sync_copy(xyz_hbm, xyz_v.at[pl.ds(0, M)])
    for u in range((MPAD - M) // 16):
        xyz_v[pl.ds(M + u * 16, 16)] = zeros16

    @pl.when(wid != LASTW)
    def _():
        pltpu.sync_copy(nbr_hbm.at[pl.ds(wid * SLAB, SLAB)], nbr_v)
        pltpu.sync_copy(w_hbm.at[pl.ds(wid * SLAB, SLAB)], w_v)
        pltpu.sync_copy(nn_hbm.at[pl.ds(wid * VPW, VPW)], nn_v)
        pltpu.sync_copy(area_hbm.at[pl.ds(wid * VPW, VPW)], area_v)

    @pl.when(wid == LASTW)
    def _():
        pltpu.sync_copy(nbr_hbm.at[pl.ds(LASTW * SLAB, TAILE)],
                        nbr_v.at[pl.ds(0, TAILE)])
        pltpu.sync_copy(w_hbm.at[pl.ds(LASTW * SLAB, TAILE)],
                        w_v.at[pl.ds(0, TAILE)])
        pltpu.sync_copy(nn_hbm.at[pl.ds(LASTW * VPW, TAILV)],
                        nn_v.at[pl.ds(0, TAILV)])
        pltpu.sync_copy(area_hbm.at[pl.ds(LASTW * VPW, TAILV)],
                        area_v.at[pl.ds(0, TAILV)])

        def _ztail(z, _):
            nbr_v[pl.ds(TAILE + z * 16, 16)] = izeros16
            return 0

        lax.fori_loop(0, (SLAB - TAILE) // 16, _ztail, 0)
        for u in range((VPW - TAILV) // 16):
            nn_v[pl.ds(TAILV + u * 16, 16)] = izeros16

    iota16 = lax.iota(jnp.int32, 16)

    def _block(b, carry):
        v0 = b * 16
        g0 = wid * VPW + v0
        sidx = 3 * g0 + 3 * iota16
        px = plsc.load_gather(xyz_v, [sidx])
        py = plsc.load_gather(xyz_v, [sidx + 1])
        pz = plsc.load_gather(xyz_v, [sidx + 2])
        nnv = nn_v[pl.ds(v0, 16)]
        areav = area_v[pl.ds(v0, 16)]
        rowbase = (v0 + iota16) * K
        for k in range(K):
            nbr = plsc.load_gather(nbr_v, [rowbase + k])
            w = plsc.load_gather(w_v, [rowbase + k])
            wf = jnp.where(nnv > k, w * areav, 0.0)
            jb = nbr * 3
            qx = plsc.load_gather(xyz_v, [jb])
            qy = plsc.load_gather(xyz_v, [jb + 1])
            qz = plsc.load_gather(xyz_v, [jb + 2])
            dx = px - qx
            dy = py - qy
            dz = pz - qz
            o = k * VPW + v0
            s0_v[pl.ds(o, 16)] = dx * dx + dy * dy + dz * dz
            nbrt_v[pl.ds(o, 16)] = nbr
            wf_v[pl.ds(o, 16)] = wf
        return carry

    lax.fori_loop(0, VPW // 16, _block, 0)
    pltpu.sync_copy(s0_v, s0_hbm.at[wid])
    pltpu.sync_copy(nbrt_v, nbro_hbm.at[wid])
    pltpu.sync_copy(wf_v, wo_hbm.at[wid])


@functools.partial(
    pl.kernel,
    out_type=[
        jax.ShapeDtypeStruct((NW, MPAD), jnp.float32),
        jax.ShapeDtypeStruct((NW, 16), jnp.float32),
    ],
    mesh=_mesh,
    scratch_types=[
        pltpu.VMEM((MPAD,), jnp.float32),
        pltpu.VMEM((MPAD,), jnp.float32),
        pltpu.VMEM((SLAB,), jnp.float32),
        pltpu.VMEM((SLAB,), jnp.int32),
        pltpu.VMEM((SLAB,), jnp.float32),
        pltpu.VMEM((16,), jnp.float32),
    ],
    compiler_params=_sc_params,
)
def _edge_call(recon_hbm, s0_hbm, nbr_hbm, w_hbm,
               gpart_hbm, epart_hbm,
               recon_v, grad_v, s0_v, nbr_v, w_v, e_v):
    wid = lax.axis_index("s") * 2 + lax.axis_index("c")

    zeros16 = jnp.zeros((16,), jnp.float32)

    pltpu.sync_copy(recon_hbm, recon_v)
    pltpu.sync_copy(s0_hbm.at[wid], s0_v)
    pltpu.sync_copy(nbr_hbm.at[wid], nbr_v)
    pltpu.sync_copy(w_hbm.at[wid], w_v)

    def _zero(z, _):
        base = z * 256
        for u in range(16):
            grad_v[pl.ds(base + u * 16, 16)] = zeros16
        return 0

    lax.fori_loop(0, MPAD // 256, _zero, 0)

    iota16 = lax.iota(jnp.int32, 16)

    def _block(b, eacc):
        v0 = b * 16
        g0 = wid * VPW + v0
        sidx = 3 * g0 + 3 * iota16
        sx = plsc.load_gather(recon_v, [sidx])
        sy = plsc.load_gather(recon_v, [sidx + 1])
        sz = plsc.load_gather(recon_v, [sidx + 2])

        gx = zeros16
        gy = zeros16
        gz = zeros16
        ek = zeros16
        for k in range(K):
            o = k * VPW + v0
            nbr = nbr_v[pl.ds(o, 16)]
            wf = w_v[pl.ds(o, 16)]
            s0 = s0_v[pl.ds(o, 16)]
            jb = nbr * 3
            rx = plsc.load_gather(recon_v, [jb])
            ry = plsc.load_gather(recon_v, [jb + 1])
            rz = plsc.load_gather(recon_v, [jb + 2])
            e1x = sx - rx
            e1y = sy - ry
            e1z = sz - rz
            d = (e1x * e1x + e1y * e1y + e1z * e1z) - s0
            wmd = wf * d
            ek = ek + wmd * d
            q = wmd * SCALE_GRAD
            cx = q * e1x
            cy = q * e1y
            cz = q * e1z
            gx = gx + cx
            gy = gy + cy
            gz = gz + cz
            plsc.addupdate_scatter(grad_v, [jb], -cx)
            plsc.addupdate_scatter(grad_v, [jb + 1], -cy)
            plsc.addupdate_scatter(grad_v, [jb + 2], -cz)

        plsc.addupdate_scatter(grad_v, [sidx], gx)
        plsc.addupdate_scatter(grad_v, [sidx + 1], gy)
        plsc.addupdate_scatter(grad_v, [sidx + 2], gz)
        return eacc + ek

    eacc = lax.fori_loop(0, VPW // 16, _block, zeros16)
    e_v[...] = eacc
    pltpu.sync_copy(grad_v, gpart_hbm.at[wid])
    pltpu.sync_copy(e_v, epart_hbm.at[wid])


def _bwd_body(wt_ref, gp_ref, ep_ref, gc_ref, e_ref):
    t = pl.program_id(0)

    @pl.when(t == 0)
    def _():
        gc_ref[...] = jnp.zeros_like(gc_ref)
        e_ref[...] = (jnp.sum(ep_ref[...]) * SCALE_E).reshape(1, 1)

    row = t * TILE + lax.broadcasted_iota(jnp.int32, (TILE, 1), 0)
    wm = jnp.where(row < M, wt_ref[...], 0.0)
    g = jnp.sum(gp_ref[...], axis=0, keepdims=True)
    contrib = lax.dot_general(g, wm, (((1,), (0,)), ((), ())),
                              preferred_element_type=jnp.float32)
    gc_ref[...] += contrib


_bwd_call = pl.pallas_call(
    _bwd_body,
    grid=(GRID,),
    in_specs=[
        pl.BlockSpec((TILE, LATENT), lambda t: (t, 0)),
        pl.BlockSpec((NW, TILE), lambda t: (0, t)),
        pl.BlockSpec((NW, 16), lambda t: (0, 0)),
    ],
    out_specs=[
        pl.BlockSpec((1, LATENT), lambda t: (0, 0)),
        pl.BlockSpec((1, 1), lambda t: (0, 0)),
    ],
    out_shape=[
        jax.ShapeDtypeStruct((1, LATENT), jnp.float32),
        jax.ShapeDtypeStruct((1, 1), jnp.float32),
    ],
)


def kernel(code, W_dec, b_dec, xyz1, neighbors, num_neighbors, weights, area):
    Wt = W_dec.T
    xyzf = xyz1.reshape(M)
    nbrF = neighbors.astype(jnp.int32).reshape(NK)
    wF = weights.reshape(NK)
    nnI = num_neighbors.astype(jnp.int32)

    s0, nbrS, wS = _pre_call(xyzf, nbrF, wF, nnI, area)

    b_pad = jnp.pad(b_dec, (0, MPAD - M)).reshape(1, MPAD)
    recon = _fwd_call(code.reshape(1, LATENT), Wt, b_pad).reshape(MPAD)

    gpart, epart = _edge_call(recon, s0, nbrS, wS)

    gc, e = _bwd_call(Wt, gpart, epart)
    return e[0, 0], gc[0]
